# baseline (speedup 1.0000x reference)
import functools

import jax
import jax.numpy as jnp
from jax import lax
from jax.experimental import pallas as pl
from jax.experimental.pallas import tpu as pltpu
from jax.experimental.pallas import tpu_sc as plsc

B = 4096
F = 26
V = 1000
D = 128
H1, H2, H3 = 1024, 512, 256
WIDE = 1024
DEEP_DIM = F * D

NC, NS = 2, 16
NW = NC * NS
NCH = F

BT = 512
SLICES = 2


def _gather_sc(table, idx, rw):
    mesh = plsc.VectorSubcoreMesh(
        core_axis_name="c", subcore_axis_name="s",
        num_cores=NC, num_subcores=NS)

    @functools.partial(
        pl.kernel,
        out_type=jax.ShapeDtypeStruct((NW * rw, DEEP_DIM), jnp.float32),
        mesh=mesh,
        scratch_types=[
            pltpu.VMEM((F, rw), jnp.int32),
            pltpu.VMEM((rw, D), jnp.float32),
            pltpu.VMEM((rw, D), jnp.float32),
            pltpu.SemaphoreType.DMA,
            pltpu.SemaphoreType.DMA,
            pltpu.SemaphoreType.DMA,
            pltpu.SemaphoreType.DMA,
        ],
    )
    def k(table_hbm, idx_hbm, out_hbm, idx_v, buf_a, buf_b, sga, sgb, ssa, ssb):
        wid = lax.axis_index("s") * NC + lax.axis_index("c")
        row0 = pl.multiple_of(wid * rw, rw)
        pltpu.sync_copy(idx_hbm.at[wid], idx_v)

        def g_start(c, buf, sem):
            pltpu.make_async_copy(table_hbm.at[idx_v.at[c]], buf, sem).start()

        def g_wait(c, buf, sem):
            pltpu.make_async_copy(table_hbm.at[idx_v.at[c]], buf, sem).wait()

        def out_block(c):
            col = pl.multiple_of(c * D, D)
            return out_hbm.at[pl.ds(row0, rw), pl.ds(col, D)]

        def s_start(c, buf, sem):
            pltpu.make_async_copy(buf, out_block(c), sem).start()

        def s_wait(c, buf, sem):
            pltpu.make_async_copy(buf, out_block(c), sem).wait()

        g_start(0, buf_a, sga)
        g_start(1, buf_b, sgb)

        def body(i, carry):
            c0 = 2 * i
            c1 = 2 * i + 1
            g_wait(c0, buf_a, sga)
            s_start(c0, buf_a, ssa)
            g_wait(c1, buf_b, sgb)
            s_start(c1, buf_b, ssb)
            s_wait(c0, buf_a, ssa)

            @pl.when(i < NCH // 2 - 1)
            def _():
                g_start(c0 + 2, buf_a, sga)

            s_wait(c1, buf_b, ssb)

            @pl.when(i < NCH // 2 - 1)
            def _():
                g_start(c1 + 2, buf_b, sgb)

            return carry

        lax.fori_loop(0, NCH // 2, body, 0)

    return k(table, idx)


def _mlp_body(xw_ref, emb_ref, w1_ref, b1_ref, w2_ref, b2_ref, w3_ref,
              b3_ref, wfcw_ref, wfcd_ref, bfc_ref, out_ref):
    h = jnp.dot(emb_ref[...].astype(jnp.bfloat16), w1_ref[...],
                preferred_element_type=jnp.float32)
    h = jnp.maximum(h + b1_ref[...], 0.0).astype(jnp.bfloat16)
    h = jnp.dot(h, w2_ref[...], preferred_element_type=jnp.float32)
    h = jnp.maximum(h + b2_ref[...], 0.0).astype(jnp.bfloat16)
    h = jnp.dot(h, w3_ref[...], preferred_element_type=jnp.float32)
    h = jnp.maximum(h + b3_ref[...], 0.0)
    out = jnp.dot(xw_ref[...], wfcw_ref[...], preferred_element_type=jnp.float32)
    out = out + jnp.dot(h, wfcd_ref[...], preferred_element_type=jnp.float32)
    out_ref[...] = out + bfc_ref[...]


def _mlp_tc(xw, emb, w1, b1, w2, b2, w3, b3, wfcw, wfcd, bfc):
    bs = xw.shape[0]
    grid = (bs // BT,)
    return pl.pallas_call(
        _mlp_body,
        grid=grid,
        in_specs=[
            pl.BlockSpec((BT, WIDE), lambda i: (i, 0)),
            pl.BlockSpec((BT, DEEP_DIM), lambda i: (i, 0)),
            pl.BlockSpec((DEEP_DIM, H1), lambda i: (0, 0)),
            pl.BlockSpec((1, H1), lambda i: (0, 0)),
            pl.BlockSpec((H1, H2), lambda i: (0, 0)),
            pl.BlockSpec((1, H2), lambda i: (0, 0)),
            pl.BlockSpec((H2, H3), lambda i: (0, 0)),
            pl.BlockSpec((1, H3), lambda i: (0, 0)),
            pl.BlockSpec((WIDE, 1), lambda i: (0, 0)),
            pl.BlockSpec((H3, 1), lambda i: (0, 0)),
            pl.BlockSpec((1, 1), lambda i: (0, 0)),
        ],
        out_specs=pl.BlockSpec((BT, 1), lambda i: (i, 0)),
        out_shape=jax.ShapeDtypeStruct((bs, 1), jnp.float32),
    )(xw, emb, w1, b1, w2, b2, w3, b3, wfcw, wfcd, bfc)


def kernel(X_w, X_d, emb_tables, W1, b1, W2, b2, W3, b3, Wfc, bfc):
    table = emb_tables.reshape(F * V, D)
    bs = B // SLICES
    rw = bs // NW
    idx = X_d.astype(jnp.int32).reshape(SLICES, NW, rw, F).transpose(0, 1, 3, 2)
    idx = idx + (jnp.arange(F, dtype=jnp.int32) * V)[None, None, :, None]
    w1 = W1.astype(jnp.bfloat16)
    w2 = W2.astype(jnp.bfloat16)
    w3 = W3.astype(jnp.bfloat16)
    outs = []
    for s in range(SLICES):
        emb = _gather_sc(table, idx[s], rw)
        outs.append(_mlp_tc(
            X_w[s * bs:(s + 1) * bs], emb,
            w1, b1.reshape(1, H1),
            w2, b2.reshape(1, H2),
            w3, b3.reshape(1, H3),
            Wfc[:WIDE], Wfc[WIDE:], bfc.reshape(1, 1),
        ))
    return jnp.concatenate(outs, axis=0) if SLICES > 1 else outs[0]


# speedup vs baseline: 1.1324x; 1.1324x over previous
import functools

import jax
import jax.numpy as jnp
from jax import lax
from jax.experimental import pallas as pl
from jax.experimental.pallas import tpu as pltpu
from jax.experimental.pallas import tpu_sc as plsc

B = 4096
F = 26
V = 1000
D = 128
H1, H2, H3 = 1024, 512, 256
WIDE = 1024
DEEP_DIM = F * D

NC, NS = 2, 16
NW = NC * NS
NCH = F

BT = 512
SLICES = 2


GW = 128


def _gather_sc(table, idx, rw):
    fpc = GW // rw
    ng = F * rw // GW
    mesh = plsc.VectorSubcoreMesh(
        core_axis_name="c", subcore_axis_name="s",
        num_cores=NC, num_subcores=NS)

    @functools.partial(
        pl.kernel,
        out_type=jax.ShapeDtypeStruct((NW * rw, DEEP_DIM), jnp.float32),
        mesh=mesh,
        scratch_types=[
            pltpu.VMEM((ng, GW), jnp.int32),
            pltpu.VMEM((GW, D), jnp.float32),
            pltpu.VMEM((GW, D), jnp.float32),
            pltpu.SemaphoreType.DMA,
            pltpu.SemaphoreType.DMA,
            pltpu.SemaphoreType.DMA,
            pltpu.SemaphoreType.DMA,
        ],
    )
    def k(table_hbm, idx_hbm, out_hbm, idx_v, buf_a, buf_b, sga, sgb, ssa, ssb):
        wid = lax.axis_index("s") * NC + lax.axis_index("c")
        row0 = pl.multiple_of(wid * rw, rw)
        pltpu.sync_copy(idx_hbm.at[wid], idx_v)

        def g_start(c, buf, sem):
            pltpu.make_async_copy(table_hbm.at[idx_v.at[c]], buf, sem).start()

        def g_wait(c, buf, sem):
            pltpu.make_async_copy(table_hbm.at[idx_v.at[c]], buf, sem).wait()

        def s_copies(c, buf, sem):
            copies = []
            for j in range(fpc):
                col = pl.multiple_of((c * fpc + j) * D, D)
                copies.append(pltpu.make_async_copy(
                    buf.at[pl.ds(j * rw, rw)],
                    out_hbm.at[pl.ds(row0, rw), pl.ds(col, D)], sem))
            return copies

        def s_start(c, buf, sem):
            for cp in s_copies(c, buf, sem):
                cp.start()

        def s_wait(c, buf, sem):
            for cp in s_copies(c, buf, sem):
                cp.wait()

        g_start(0, buf_a, sga)
        g_start(1, buf_b, sgb)

        def body(i, carry):
            c0 = 2 * i
            c1 = 2 * i + 1
            g_wait(c0, buf_a, sga)
            s_start(c0, buf_a, ssa)

            @pl.when(c1 < ng)
            def _():
                g_wait(c1, buf_b, sgb)
                s_start(c1, buf_b, ssb)

            s_wait(c0, buf_a, ssa)

            @pl.when(c0 + 2 < ng)
            def _():
                g_start(c0 + 2, buf_a, sga)

            @pl.when(c1 < ng)
            def _():
                s_wait(c1, buf_b, ssb)

                @pl.when(c1 + 2 < ng)
                def _():
                    g_start(c1 + 2, buf_b, sgb)

            return carry

        lax.fori_loop(0, (ng + 1) // 2, body, 0)

    return k(table, idx)


def _mlp_body(xw_ref, emb_ref, w1_ref, b1_ref, w2_ref, b2_ref, w3_ref,
              b3_ref, wfcw_ref, wfcd_ref, bfc_ref, out_ref):
    h = jnp.dot(emb_ref[...].astype(jnp.bfloat16), w1_ref[...],
                preferred_element_type=jnp.float32)
    h = jnp.maximum(h + b1_ref[...], 0.0).astype(jnp.bfloat16)
    h = jnp.dot(h, w2_ref[...], preferred_element_type=jnp.float32)
    h = jnp.maximum(h + b2_ref[...], 0.0).astype(jnp.bfloat16)
    h = jnp.dot(h, w3_ref[...], preferred_element_type=jnp.float32)
    h = jnp.maximum(h + b3_ref[...], 0.0)
    out = jnp.dot(xw_ref[...], wfcw_ref[...], preferred_element_type=jnp.float32)
    out = out + jnp.dot(h, wfcd_ref[...], preferred_element_type=jnp.float32)
    out_ref[...] = out + bfc_ref[...]


def _mlp_tc(xw, emb, w1, b1, w2, b2, w3, b3, wfcw, wfcd, bfc, off_bt):
    bs = emb.shape[0]
    grid = (bs // BT,)
    return pl.pallas_call(
        _mlp_body,
        grid=grid,
        in_specs=[
            pl.BlockSpec((BT, WIDE), lambda i: (i + off_bt, 0)),
            pl.BlockSpec((BT, DEEP_DIM), lambda i: (i, 0)),
            pl.BlockSpec((DEEP_DIM, H1), lambda i: (0, 0)),
            pl.BlockSpec((1, H1), lambda i: (0, 0)),
            pl.BlockSpec((H1, H2), lambda i: (0, 0)),
            pl.BlockSpec((1, H2), lambda i: (0, 0)),
            pl.BlockSpec((H2, H3), lambda i: (0, 0)),
            pl.BlockSpec((1, H3), lambda i: (0, 0)),
            pl.BlockSpec((WIDE, 1), lambda i: (0, 0)),
            pl.BlockSpec((H3, 1), lambda i: (0, 0)),
            pl.BlockSpec((1, 1), lambda i: (0, 0)),
        ],
        out_specs=pl.BlockSpec((BT, 1), lambda i: (i, 0)),
        out_shape=jax.ShapeDtypeStruct((bs, 1), jnp.float32),
    )(xw, emb, w1, b1, w2, b2, w3, b3, wfcw, wfcd, bfc)


def kernel(X_w, X_d, emb_tables, W1, b1, W2, b2, W3, b3, Wfc, bfc):
    table = emb_tables.reshape(F * V, D)
    bs = B // SLICES
    rw = bs // NW
    ng = F * rw // GW
    idx = X_d.astype(jnp.int32).reshape(SLICES, NW, rw, F).transpose(0, 1, 3, 2)
    idx = idx + (jnp.arange(F, dtype=jnp.int32) * V)[None, None, :, None]
    idx = idx.reshape(SLICES, NW, ng, GW)
    w1 = W1.astype(jnp.bfloat16)
    w2 = W2.astype(jnp.bfloat16)
    w3 = W3.astype(jnp.bfloat16)
    outs = []
    for s in range(SLICES):
        emb = _gather_sc(table, idx[s], rw)
        outs.append(_mlp_tc(
            X_w, emb,
            w1, b1.reshape(1, H1),
            w2, b2.reshape(1, H2),
            w3, b3.reshape(1, H3),
            Wfc[:WIDE], Wfc[WIDE:], bfc.reshape(1, 1),
            s * (bs // BT),
        ))
    return jnp.concatenate(outs, axis=0) if SLICES > 1 else outs[0]
